# Initial kernel scaffold; baseline (speedup 1.0000x reference)
import jax
import jax.numpy as jnp
from jax.experimental import pallas as pl


def kernel(node_list, features, W, b):
    raise NotImplementedError("write your pallas kernel here")


# speedup vs baseline: 10.8065x; 10.8065x over previous
import functools

import jax
import jax.numpy as jnp
from jax import lax
from jax.experimental import pallas as pl
from jax.experimental.pallas import tpu as pltpu
from jax.experimental.pallas import tpu_sc as plsc

WIN = 33


def _make_window_kernel(R, D, EMB, NB):
    def body(a_ref, c_ref, wt_ref, b_ref, o_ref):
        f = jnp.concatenate([a_ref[...], c_ref[...]], axis=0)
        h = jnp.dot(f, wt_ref[...], precision=lax.Precision.HIGHEST,
                    preferred_element_type=jnp.float32)
        g = jnp.tanh(h + b_ref[...])
        acc = g[0:R]
        for k in range(1, WIN):
            acc = acc + g[k:k + R]
        sc = acc * (1.0 / WIN)
        o_ref[...] = jnp.concatenate([sc, sc, sc, sc], axis=1)

    return pl.pallas_call(
        body,
        grid=(NB,),
        in_specs=[
            pl.BlockSpec((R, D), lambda i: (i, 0)),
            pl.BlockSpec((32, D), lambda i: (i * (R // 32) + R // 32, 0)),
            pl.BlockSpec((D, EMB), lambda i: (0, 0)),
            pl.BlockSpec((1, EMB), lambda i: (0, 0)),
        ],
        out_specs=pl.BlockSpec((R, 128), lambda i: (i, 0)),
        out_shape=jax.ShapeDtypeStruct((NB * R, 128), jnp.float32),
    )


def _make_sc_gather(B, NW, CH):
    mesh = plsc.VectorSubcoreMesh(core_axis_name="c", subcore_axis_name="s")

    @functools.partial(
        pl.kernel,
        mesh=mesh,
        out_type=jax.ShapeDtypeStruct((B, 128), jnp.float32),
        scratch_types=[
            pltpu.VMEM((CH, 128), jnp.int32),
            pltpu.VMEM((CH * 128, 128), jnp.float32),
            pltpu.SemaphoreType.DMA,
        ],
    )
    def sc_gather(idx_hbm, tbl_hbm, out_hbm, idx_v, rows_v, sem):
        wid = lax.axis_index("s") * 2 + lax.axis_index("c")
        pltpu.sync_copy(idx_hbm.at[pl.ds(wid * CH, CH)], idx_v)
        copies = []
        for j in range(CH):
            copies.append(
                pltpu.async_copy(tbl_hbm.at[idx_v.at[j]],
                                 rows_v.at[pl.ds(j * 128, 128)], sem))
        for c in copies:
            c.wait()
        pltpu.sync_copy(rows_v, out_hbm.at[pl.ds(wid * CH * 128, CH * 128)])

    return sc_gather


def kernel(node_list, features, W, b):
    N, D = features.shape
    EMB = W.shape[0]
    B = node_list.shape[0]

    R = 2048
    NB = -(-N // R)
    NP = NB * R

    feat_ext = jnp.concatenate(
        [features, features[:WIN - 1],
         jnp.zeros((NP - N, D), features.dtype)], axis=0)

    ws = _make_window_kernel(R, D, EMB, NB)(
        feat_ext, feat_ext, W.T, b.reshape(1, EMB))

    NW = 32
    CH = B // NW // 128
    idx2 = node_list.reshape(B // 128, 128)
    gathered = _make_sc_gather(B, NW, CH)(idx2, ws)
    return gathered[:, :EMB]


# speedup vs baseline: 21.2687x; 1.9681x over previous
import functools

import jax
import jax.numpy as jnp
from jax import lax
from jax.experimental import pallas as pl
from jax.experimental.pallas import tpu as pltpu
from jax.experimental.pallas import tpu_sc as plsc

WIN = 33


def _make_window_kernel(R, D, EMB, NB):
    def body(a_ref, c1_ref, c2_ref, wt_ref, b_ref, o_ref):
        f = jnp.concatenate([a_ref[...], c1_ref[...], c2_ref[...]], axis=0)
        h = jnp.dot(f, wt_ref[...], precision=lax.Precision.HIGHEST,
                    preferred_element_type=jnp.float32)
        g = jnp.tanh(h + b_ref[...])
        a = g[0:R + 8] + g[8:R + 16] + g[16:R + 24] + g[24:R + 32]
        bb = a[0:R + 7] + a[1:R + 8]
        c = bb[0:R + 5] + bb[2:R + 7]
        d = c[0:R] + c[4:R + 4]
        sc = (d + g[32:R + 32]) * (1.0 / WIN)
        o_ref[...] = jnp.concatenate([sc, sc, sc, sc], axis=1)

    NH = (NB * R) // 16
    return pl.pallas_call(
        body,
        grid=(NB,),
        in_specs=[
            pl.BlockSpec((R, D), lambda i: (i, 0)),
            pl.BlockSpec((16, D), lambda i: (lax.rem((i + 1) * (R // 16), NH), 0)),
            pl.BlockSpec((16, D), lambda i: (lax.rem((i + 1) * (R // 16) + 1, NH), 0)),
            pl.BlockSpec((D, EMB), lambda i: (0, 0)),
            pl.BlockSpec((1, EMB), lambda i: (0, 0)),
        ],
        out_specs=pl.BlockSpec((R, 128), lambda i: (i, 0)),
        out_shape=jax.ShapeDtypeStruct((NB * R, 128), jnp.float32),
    )


def _make_sc_gather(B, NW, CH):
    mesh = plsc.VectorSubcoreMesh(core_axis_name="c", subcore_axis_name="s")

    @functools.partial(
        pl.kernel,
        mesh=mesh,
        out_type=jax.ShapeDtypeStruct((B, 128), jnp.float32),
        scratch_types=[
            pltpu.VMEM((CH, 128), jnp.int32),
            pltpu.VMEM((CH * 128, 128), jnp.float32),
            pltpu.SemaphoreType.DMA,
        ],
    )
    def sc_gather(idx_hbm, tbl_hbm, out_hbm, idx_v, rows_v, sem):
        wid = lax.axis_index("s") * 2 + lax.axis_index("c")
        pltpu.sync_copy(idx_hbm.at[pl.ds(wid * CH, CH)], idx_v)
        copies = []
        for j in range(CH):
            copies.append(
                pltpu.async_copy(tbl_hbm.at[idx_v.at[j]],
                                 rows_v.at[pl.ds(j * 128, 128)], sem))
        for c in copies:
            c.wait()
        pltpu.sync_copy(rows_v, out_hbm.at[pl.ds(wid * CH * 128, CH * 128)])

    return sc_gather


def kernel(node_list, features, W, b):
    N, D = features.shape
    EMB = W.shape[0]
    B = node_list.shape[0]

    R = 2000
    NB = N // R

    ws = _make_window_kernel(R, D, EMB, NB)(
        features, features, features, W.T, b.reshape(1, EMB))

    NW = 32
    CH = B // NW // 128
    idx2 = node_list.reshape(B // 128, 128)
    gathered = _make_sc_gather(B, NW, CH)(idx2, ws)
    return gathered[:, :EMB]


# speedup vs baseline: 27.1928x; 1.2785x over previous
import functools

import jax
import jax.numpy as jnp
from jax import lax
from jax.experimental import pallas as pl
from jax.experimental.pallas import tpu as pltpu
from jax.experimental.pallas import tpu_sc as plsc

WIN = 33


def _make_window_kernel(N, M, Q, D, EMB, NB):
    def body(*refs):
        a_refs = refs[0:4]
        h_refs = refs[4:20]
        wt_ref, b_ref, o_ref = refs[20], refs[21], refs[22]
        QH = Q + 32
        parts = []
        for s in range(4):
            parts.append(a_refs[s][...])
            for j in range(4):
                parts.append(h_refs[4 * s + j][...])
        f_all = jnp.concatenate(parts, axis=0)
        h = jnp.dot(f_all, wt_ref[...], preferred_element_type=jnp.float32)
        g = jnp.tanh(h + b_ref[...])
        gp = jnp.concatenate([g[s * QH:(s + 1) * QH] for s in range(4)],
                             axis=1)
        a = gp[0:Q + 8] + gp[8:Q + 16] + gp[16:Q + 24] + gp[24:Q + 32]
        bb = a[0:Q + 7] + a[1:Q + 8]
        c = bb[0:Q + 5] + bb[2:Q + 7]
        d = c[0:Q] + c[4:Q + 4]
        o_ref[...] = (d + gp[32:Q + 32]) * (1.0 / WIN)

    in_specs = []
    for s in range(4):
        in_specs.append(
            pl.BlockSpec((Q, D), functools.partial(
                lambda s_, i: (s_ * (M // Q) + i, 0), s)))
    NH8 = N // 8
    for s in range(4):
        for j in range(4):
            in_specs.append(
                pl.BlockSpec((8, D), functools.partial(
                    lambda s_, j_, i: (
                        lax.rem(s_ * (M // 8) + (i + 1) * (Q // 8) + j_, NH8),
                        0), s, j)))
    in_specs.append(pl.BlockSpec((D, EMB), lambda i: (0, 0)))
    in_specs.append(pl.BlockSpec((1, EMB), lambda i: (0, 0)))

    return pl.pallas_call(
        body,
        grid=(NB,),
        in_specs=in_specs,
        out_specs=pl.BlockSpec((Q, 128), lambda i: (i, 0)),
        out_shape=jax.ShapeDtypeStruct((M, 128), jnp.float32),
    )


def _make_sc_gather(B, M, NW, CH):
    mesh = plsc.VectorSubcoreMesh(core_axis_name="c", subcore_axis_name="s")

    BW = CH * 128

    @functools.partial(
        pl.kernel,
        mesh=mesh,
        out_type=jax.ShapeDtypeStruct((B, 128), jnp.float32),
        scratch_types=[
            pltpu.VMEM((BW,), jnp.int32),
            pltpu.VMEM((BW,), jnp.int32),
            pltpu.VMEM((BW, 128), jnp.float32),
            pltpu.SemaphoreType.DMA,
        ],
    )
    def sc_gather(idx_hbm, tbl_hbm, out_hbm, idx_v, row_v, rows_v, sem):
        wid = lax.axis_index("s") * 2 + lax.axis_index("c")
        pltpu.sync_copy(idx_hbm.at[pl.ds(wid * BW, BW)], idx_v)
        for k in range(BW // 16):
            row_v[pl.ds(k * 16, 16)] = lax.rem(idx_v[pl.ds(k * 16, 16)],
                                               jnp.int32(M))
        copies = []
        for j in range(CH):
            copies.append(
                pltpu.async_copy(tbl_hbm.at[row_v.at[pl.ds(j * 128, 128)]],
                                 rows_v.at[pl.ds(j * 128, 128)], sem))
        for c in copies:
            c.wait()
        pltpu.sync_copy(rows_v, out_hbm.at[pl.ds(wid * BW, BW)])

    return sc_gather


def _make_select_kernel(B, M, EMB, RB):
    def body(g_ref, n_ref, o_ref):
        n = n_ref[...]
        f1 = (n >= M).astype(jnp.float32)
        f2 = (n >= 2 * M).astype(jnp.float32)
        f3 = (n >= 3 * M).astype(jnp.float32)
        g = g_ref[...]
        o_ref[...] = (g[:, 0:EMB] * (1.0 - f1)
                      + g[:, EMB:2 * EMB] * (f1 - f2)
                      + g[:, 2 * EMB:3 * EMB] * (f2 - f3)
                      + g[:, 3 * EMB:4 * EMB] * f3)

    return pl.pallas_call(
        body,
        grid=(B // RB,),
        in_specs=[
            pl.BlockSpec((RB, 128), lambda i: (i, 0)),
            pl.BlockSpec((RB, 1), lambda i: (i, 0)),
        ],
        out_specs=pl.BlockSpec((RB, EMB), lambda i: (i, 0)),
        out_shape=jax.ShapeDtypeStruct((B, EMB), jnp.float32),
    )


def kernel(node_list, features, W, b):
    N, D = features.shape
    EMB = W.shape[0]
    B = node_list.shape[0]

    M = N // 4
    Q = 1000
    NB = M // Q

    ws = _make_window_kernel(N, M, Q, D, EMB, NB)(
        *([features] * 20), W.T, b.reshape(1, EMB))

    NW = 32
    CH = B // NW // 128
    gathered = _make_sc_gather(B, M, NW, CH)(node_list, ws)

    return _make_select_kernel(B, M, EMB, 2048)(
        gathered, node_list.reshape(B, 1))


# speedup vs baseline: 29.3155x; 1.0781x over previous
import functools

import jax
import jax.numpy as jnp
from jax import lax
from jax.experimental import pallas as pl
from jax.experimental.pallas import tpu as pltpu
from jax.experimental.pallas import tpu_sc as plsc

WIN = 33


def _make_window_kernel(N, M, Q, D, EMB, NB):
    def body(*refs):
        a_refs = refs[0:4]
        h_refs = refs[4:20]
        wt_ref, b_ref, o_ref = refs[20], refs[21], refs[22]
        QH = Q + 32
        parts = []
        for s in range(4):
            parts.append(a_refs[s][...])
            for j in range(4):
                parts.append(h_refs[4 * s + j][...])
        f_all = jnp.concatenate(parts, axis=0)
        h = jnp.dot(f_all, wt_ref[...], preferred_element_type=jnp.float32)
        g = jnp.tanh(h + b_ref[...])
        gp = jnp.concatenate([g[s * QH:(s + 1) * QH] for s in range(4)],
                             axis=1)
        a = gp[0:Q + 8] + gp[8:Q + 16] + gp[16:Q + 24] + gp[24:Q + 32]
        bb = a[0:Q + 7] + a[1:Q + 8]
        c = bb[0:Q + 5] + bb[2:Q + 7]
        d = c[0:Q] + c[4:Q + 4]
        o_ref[...] = (d + gp[32:Q + 32]) * (1.0 / WIN)

    in_specs = []
    for s in range(4):
        in_specs.append(
            pl.BlockSpec((Q, D), functools.partial(
                lambda s_, i: (s_ * (M // Q) + i, 0), s)))
    NH8 = N // 8
    for s in range(4):
        for j in range(4):
            in_specs.append(
                pl.BlockSpec((8, D), functools.partial(
                    lambda s_, j_, i: (
                        lax.rem(s_ * (M // 8) + (i + 1) * (Q // 8) + j_, NH8),
                        0), s, j)))
    in_specs.append(pl.BlockSpec((D, EMB), lambda i: (0, 0)))
    in_specs.append(pl.BlockSpec((1, EMB), lambda i: (0, 0)))

    return pl.pallas_call(
        body,
        grid=(NB,),
        in_specs=in_specs,
        out_specs=pl.BlockSpec((Q, 128), lambda i: (i, 0)),
        out_shape=jax.ShapeDtypeStruct((M, 128), jnp.float32),
    )


def _make_sc_gather(B, M, NW, CH):
    mesh = plsc.VectorSubcoreMesh(core_axis_name="c", subcore_axis_name="s")

    BW = CH * 128

    @functools.partial(
        pl.kernel,
        mesh=mesh,
        out_type=jax.ShapeDtypeStruct((B, 128), jnp.float32),
        scratch_types=[
            pltpu.VMEM((BW,), jnp.int32),
            pltpu.VMEM((BW,), jnp.int32),
            pltpu.VMEM((BW, 128), jnp.float32),
            pltpu.SemaphoreType.DMA,
        ],
    )
    def sc_gather(idx_hbm, tbl_hbm, out_hbm, idx_v, row_v, rows_v, sem):
        wid = lax.axis_index("s") * 2 + lax.axis_index("c")
        pltpu.sync_copy(idx_hbm.at[pl.ds(wid * BW, BW)], idx_v)
        for k in range(BW // 16):
            row_v[pl.ds(k * 16, 16)] = lax.rem(idx_v[pl.ds(k * 16, 16)],
                                               jnp.int32(M))
        copies = []
        for j in range(CH):
            copies.append(
                pltpu.async_copy(tbl_hbm.at[row_v.at[pl.ds(j * 128, 128)]],
                                 rows_v.at[pl.ds(j * 128, 128)], sem))
        for c in copies:
            c.wait()
        pltpu.sync_copy(rows_v, out_hbm.at[pl.ds(wid * BW, BW)])

    return sc_gather


def _make_select_kernel(B, M, EMB, RB):
    def body(g_ref, n_ref, o_ref):
        n = n_ref[...]
        s32 = ((n >= M).astype(jnp.int32) + (n >= 2 * M).astype(jnp.int32)
               + (n >= 3 * M).astype(jnp.int32)) * EMB
        liota = lax.broadcasted_iota(jnp.int32, (RB, 128), 1)
        m = ((liota >= s32) & (liota < s32 + EMB)).astype(jnp.float32)
        gw = g_ref[...] * m
        ri = lax.broadcasted_iota(jnp.int32, (128, EMB), 0)
        ci = lax.broadcasted_iota(jnp.int32, (128, EMB), 1)
        fold = (lax.rem(ri, EMB) == ci).astype(jnp.float32)
        o_ref[...] = jnp.dot(gw, fold, preferred_element_type=jnp.float32)

    return pl.pallas_call(
        body,
        grid=(B // RB,),
        in_specs=[
            pl.BlockSpec((RB, 128), lambda i: (i, 0)),
            pl.BlockSpec((RB, 1), lambda i: (i, 0)),
        ],
        out_specs=pl.BlockSpec((RB, EMB), lambda i: (i, 0)),
        out_shape=jax.ShapeDtypeStruct((B, EMB), jnp.float32),
    )


def kernel(node_list, features, W, b):
    N, D = features.shape
    EMB = W.shape[0]
    B = node_list.shape[0]

    M = N // 4
    Q = 1000
    NB = M // Q

    ws = _make_window_kernel(N, M, Q, D, EMB, NB)(
        *([features] * 20), W.T, b.reshape(1, EMB))

    NW = 32
    CH = B // NW // 128
    gathered = _make_sc_gather(B, M, NW, CH)(node_list, ws)

    return _make_select_kernel(B, M, EMB, 4096)(
        gathered, node_list.reshape(B, 1))


# speedup vs baseline: 33.5291x; 1.1437x over previous
import functools

import jax
import jax.numpy as jnp
from jax import lax
from jax.experimental import pallas as pl
from jax.experimental.pallas import tpu as pltpu
from jax.experimental.pallas import tpu_sc as plsc

WIN = 33


def _make_window_kernel(N, M, Q, D, EMB, NB):
    def body(*refs):
        a_refs = refs[0:4]
        h_refs = refs[4:20]
        wt_ref, b_ref, o_ref = refs[20], refs[21], refs[22]
        QH = Q + 32
        parts = []
        for s in range(4):
            parts.append(a_refs[s][...])
            for j in range(4):
                parts.append(h_refs[4 * s + j][...])
        f_all = jnp.concatenate(parts, axis=0)
        h = jnp.dot(f_all, wt_ref[...], preferred_element_type=jnp.float32)
        g = jnp.tanh(h + b_ref[...])
        gp = jnp.concatenate([g[s * QH:(s + 1) * QH] for s in range(4)],
                             axis=1)
        a = gp[0:Q + 8] + gp[8:Q + 16] + gp[16:Q + 24] + gp[24:Q + 32]
        bb = a[0:Q + 7] + a[1:Q + 8]
        c = bb[0:Q + 5] + bb[2:Q + 7]
        d = c[0:Q] + c[4:Q + 4]
        o_ref[...] = (d + gp[32:Q + 32]) * (1.0 / WIN)

    in_specs = []
    for s in range(4):
        in_specs.append(
            pl.BlockSpec((Q, D), functools.partial(
                lambda s_, i: (s_ * (M // Q) + i, 0), s)))
    NH8 = N // 8
    for s in range(4):
        for j in range(4):
            in_specs.append(
                pl.BlockSpec((8, D), functools.partial(
                    lambda s_, j_, i: (
                        lax.rem(s_ * (M // 8) + (i + 1) * (Q // 8) + j_, NH8),
                        0), s, j)))
    in_specs.append(pl.BlockSpec((D, EMB), lambda i: (0, 0)))
    in_specs.append(pl.BlockSpec((1, EMB), lambda i: (0, 0)))

    return pl.pallas_call(
        body,
        grid=(NB,),
        in_specs=in_specs,
        out_specs=pl.BlockSpec((Q, 128), lambda i: (i, 0)),
        out_shape=jax.ShapeDtypeStruct((M, 128), jnp.float32),
    )


def _make_sc_gather(B, M, NW, CH):
    mesh = plsc.VectorSubcoreMesh(core_axis_name="c", subcore_axis_name="s")

    BW = CH * 128

    @functools.partial(
        pl.kernel,
        mesh=mesh,
        out_type=jax.ShapeDtypeStruct((B, 128), jnp.float32),
        scratch_types=[
            pltpu.VMEM((BW,), jnp.int32),
            pltpu.VMEM((BW,), jnp.int32),
            pltpu.VMEM((BW, 128), jnp.float32),
            pltpu.SemaphoreType.DMA,
        ],
    )
    def sc_gather(idx_hbm, tbl_hbm, out_hbm, idx_v, row_v, rows_v, sem):
        wid = lax.axis_index("s") * 2 + lax.axis_index("c")
        pltpu.sync_copy(idx_hbm.at[pl.ds(wid * BW, BW)], idx_v)
        for k in range(BW // 16):
            row_v[pl.ds(k * 16, 16)] = lax.rem(idx_v[pl.ds(k * 16, 16)],
                                               jnp.int32(M))
        copies = []
        for j in range(CH):
            copies.append(
                pltpu.async_copy(tbl_hbm.at[row_v.at[pl.ds(j * 128, 128)]],
                                 rows_v.at[pl.ds(j * 128, 128)], sem))
        for c in copies:
            c.wait()
        pltpu.sync_copy(rows_v, out_hbm.at[pl.ds(wid * BW, BW)])

    return sc_gather


def _make_select_kernel(B, M, EMB, RB):
    def body(g_ref, n_ref, o_ref):
        n = n_ref[...]
        s32 = ((n >= M).astype(jnp.int32) + (n >= 2 * M).astype(jnp.int32)
               + (n >= 3 * M).astype(jnp.int32)) * EMB
        liota = lax.broadcasted_iota(jnp.int32, (RB, 128), 1)
        m = ((liota >= s32) & (liota < s32 + EMB)).astype(jnp.float32)
        gw = g_ref[...] * m
        ri = lax.broadcasted_iota(jnp.int32, (128, EMB), 0)
        ci = lax.broadcasted_iota(jnp.int32, (128, EMB), 1)
        fold = (lax.rem(ri, EMB) == ci).astype(jnp.float32)
        o_ref[...] = jnp.dot(gw, fold, preferred_element_type=jnp.float32)

    return pl.pallas_call(
        body,
        grid=(B // RB,),
        in_specs=[
            pl.BlockSpec((RB, 128), lambda i: (i, 0)),
            pl.BlockSpec((RB, 1), lambda i: (i, 0)),
        ],
        out_specs=pl.BlockSpec((RB, EMB), lambda i: (i, 0)),
        out_shape=jax.ShapeDtypeStruct((B, EMB), jnp.float32),
    )


def kernel(node_list, features, W, b):
    N, D = features.shape
    EMB = W.shape[0]
    B = node_list.shape[0]

    M = N // 4
    Q = 5000
    NB = M // Q

    ws = _make_window_kernel(N, M, Q, D, EMB, NB)(
        *([features] * 20), W.T, b.reshape(1, EMB))

    NW = 32
    CH = B // NW // 128
    gathered = _make_sc_gather(B, M, NW, CH)(node_list, ws)

    return _make_select_kernel(B, M, EMB, 4096)(
        gathered, node_list.reshape(B, 1))
